# Initial kernel scaffold; baseline (speedup 1.0000x reference)
import jax
import jax.numpy as jnp
from jax.experimental import pallas as pl


def kernel(msa, pair, xyz, state, seq1hot, idx, w_seq_vec, W_x, b_x, W_e1, b_e1, W_e2, b_e2, W_msg, b_msg, W_out, b_out):
    raise NotImplementedError("write your pallas kernel here")


# speedup vs baseline: 30.4704x; 30.4704x over previous
import jax
import jax.numpy as jnp
from jax.experimental import pallas as pl
from jax.experimental.pallas import tpu as pltpu

L = 512
N_MSA = 64
D_PAIR = 128
BI = 32
NT = L // BI
TOPK = 64
NB_RBF = 36


def _ln(x, eps=1e-5):
    m = x.mean(-1, keepdims=True)
    v = ((x - m) ** 2).mean(-1, keepdims=True)
    return (x - m) / jnp.sqrt(v + eps)


def _node_body(msa_ref, state_ref, seq_ref, wseq_ref, wxm_ref, wxs_ref,
               wxst_ref, bx_ref, node_ref):
    msa = msa_ref[...]
    mn = _ln(msa)
    wv = wseq_ref[...].reshape(1, 1, -1)
    logits = jnp.sum(mn * wv, axis=-1)
    mx = jnp.max(logits, axis=0, keepdims=True)
    e = jnp.exp(logits - mx)
    w = e / jnp.sum(e, axis=0, keepdims=True)
    nm = jnp.sum(w[..., None] * mn, axis=0)
    st = _ln(state_ref[...])
    pre = (jnp.dot(nm, wxm_ref[...], preferred_element_type=jnp.float32)
           + jnp.dot(seq_ref[...], wxs_ref[...], preferred_element_type=jnp.float32)
           + jnp.dot(st, wxst_ref[...], preferred_element_type=jnp.float32)
           + bx_ref[...])
    node_ref[...] = _ln(pre)


def _edge_body(pair_ref, node_ref,
               cax_ref, cay_ref, caz_ref, caxt_ref, cayt_ref, cazt_ref,
               idxr_ref, idxc_ref,
               we1_ref, be1_ref, we2pe_ref, we2rbf_ref, we2nb_ref, be2_ref,
               wms_ref, wmt_ref, wme_ref, wmd_ref, bmsg_ref,
               won_ref, woa_ref, bout_ref,
               out_ref, agg_sc):
    t = pl.program_id(0)

    @pl.when(t == 0)
    def _init():
        agg_sc[...] = jnp.zeros_like(agg_sc)

    pairb = pair_ref[...]
    pn = _ln(pairb)
    pe1 = _ln(jnp.dot(pn.reshape(BI * L, D_PAIR), we1_ref[...],
                      preferred_element_type=jnp.float32).reshape(BI, L, 32)
              + be1_ref[...].reshape(1, 1, 32))

    dvx = caxt_ref[...] - cax_ref[...]
    dvy = cayt_ref[...] - cay_ref[...]
    dvz = cazt_ref[...] - caz_ref[...]
    d2 = dvx * dvx + dvy * dvy + dvz * dvz
    D = jnp.sqrt(jnp.maximum(d2, 1e-12))

    mu = jax.lax.broadcasted_iota(jnp.int32, (1, 1, NB_RBF), 2).astype(
        jnp.float32) * (20.0 / 35.0)
    sg = 20.0 / NB_RBF
    rb = jnp.exp(-(((D[..., None] - mu) / sg) ** 2))

    sep = idxc_ref[...] - idxr_ref[...]
    nb = jnp.sign(sep) * jnp.log(jnp.abs(sep) + 1.0)

    pe2 = _ln(jnp.dot(pe1.reshape(BI * L, 32), we2pe_ref[...],
                      preferred_element_type=jnp.float32).reshape(BI, L, 32)
              + jnp.dot(rb.reshape(BI * L, NB_RBF), we2rbf_ref[...],
                        preferred_element_type=jnp.float32).reshape(BI, L, 32)
              + nb[..., None] * we2nb_ref[...].reshape(1, 1, 32)
              + be2_ref[...].reshape(1, 1, 32))

    rowid = t * BI + jax.lax.broadcasted_iota(jnp.int32, (BI, L), 0)
    colid = jax.lax.broadcasted_iota(jnp.int32, (BI, L), 1)
    diag = rowid == colid
    Dm = jnp.where(diag, D + 999.9, D)
    bits = jax.lax.bitcast_convert_type(Dm, jnp.int32)

    def bs_body(_, carry):
        lo, hi = carry
        mid = lo + (hi - lo) // 2
        cnt = jnp.sum((bits <= mid).astype(jnp.int32), axis=1, keepdims=True)
        ge = cnt >= TOPK
        return jnp.where(ge, lo, mid + 1), jnp.where(ge, mid, hi)

    lo0 = jnp.zeros((BI, 1), jnp.int32)
    hi0 = jnp.full((BI, 1), 0x7F000000, jnp.int32)
    _, thr = jax.lax.fori_loop(0, 31, bs_body, (lo0, hi0))
    topk_mask = bits <= thr
    seql = (jnp.abs(sep) < 9.0) & (~diag)
    mask = topk_mask | seql

    node = node_ref[...]
    nodei = node_ref[pl.ds(t * BI, BI), :]
    a_i = jnp.dot(nodei, wms_ref[...], preferred_element_type=jnp.float32)
    b_j = jnp.dot(node, wmt_ref[...], preferred_element_type=jnp.float32)
    c_ij = jnp.dot(pe2.reshape(BI * L, 32), wme_ref[...],
                   preferred_element_type=jnp.float32).reshape(BI, L, 32)
    wmd = wmd_ref[...]
    dterm = (dvx[..., None] * wmd[0:1, :].reshape(1, 1, 32)
             + dvy[..., None] * wmd[1:2, :].reshape(1, 1, 32)
             + dvz[..., None] * wmd[2:3, :].reshape(1, 1, 32))
    msg = a_i[:, None, :] + b_j[None, :, :] + c_ij + dterm \
        + bmsg_ref[...].reshape(1, 1, 32)
    msg = jnp.maximum(msg, 0.0)
    maskf = jnp.where(mask, 1.0, 0.0)
    msg = msg * maskf[..., None]
    agg_sc[...] += jnp.sum(msg, axis=0)

    @pl.when(t == NT - 1)
    def _fin():
        out_ref[...] = (jnp.dot(node, won_ref[...], preferred_element_type=jnp.float32)
                        + jnp.dot(agg_sc[...], woa_ref[...], preferred_element_type=jnp.float32)
                        + bout_ref[...])


def kernel(msa, pair, xyz, state, seq1hot, idx, w_seq_vec, W_x, b_x,
           W_e1, b_e1, W_e2, b_e2, W_msg, b_msg, W_out, b_out):
    f32 = jnp.float32
    msa2 = msa[0]
    pair2 = pair[0]
    state2 = state[0]
    seq2 = seq1hot[0]
    ca = xyz[0, :, 1]
    cax = ca[:, 0:1]
    cay = ca[:, 1:2]
    caz = ca[:, 2:3]
    caxt = cax.reshape(1, L)
    cayt = cay.reshape(1, L)
    cazt = caz.reshape(1, L)
    idxf = idx[0].astype(f32)
    idxr = idxf.reshape(L, 1)
    idxc = idxf.reshape(1, L)

    full = lambda shp: pl.BlockSpec(shp, lambda t: tuple(0 for _ in shp))
    BL = 128
    node = pl.pallas_call(
        _node_body,
        grid=(L // BL,),
        in_specs=[
            pl.BlockSpec((N_MSA, BL, 64), lambda t: (0, t, 0)),
            pl.BlockSpec((BL, 16), lambda t: (t, 0)),
            pl.BlockSpec((BL, 21), lambda t: (t, 0)),
            full((1, 64)), full((64, 32)), full((21, 32)),
            full((16, 32)), full((1, 32)),
        ],
        out_specs=pl.BlockSpec((BL, 32), lambda t: (t, 0)),
        out_shape=jax.ShapeDtypeStruct((L, 32), f32),
        compiler_params=pltpu.CompilerParams(
            dimension_semantics=("arbitrary",)),
    )(msa2, state2, seq2, w_seq_vec.reshape(1, -1),
      W_x[:64], W_x[64:85], W_x[85:], b_x.reshape(1, -1))
    out = pl.pallas_call(
        _edge_body,
        grid=(NT,),
        in_specs=[
            pl.BlockSpec((BI, L, D_PAIR), lambda t: (t, 0, 0)),
            full((L, 32)),
            pl.BlockSpec((BI, 1), lambda t: (t, 0)),
            pl.BlockSpec((BI, 1), lambda t: (t, 0)),
            pl.BlockSpec((BI, 1), lambda t: (t, 0)),
            full((1, L)), full((1, L)), full((1, L)),
            pl.BlockSpec((BI, 1), lambda t: (t, 0)),
            full((1, L)),
            full((D_PAIR, 32)), full((1, 32)),
            full((32, 32)), full((NB_RBF, 32)), full((1, 32)),
            full((1, 32)),
            full((32, 32)), full((32, 32)), full((32, 32)),
            full((3, 32)), full((1, 32)),
            full((32, 16)), full((32, 16)), full((1, 16)),
        ],
        out_specs=pl.BlockSpec((L, 16), lambda t: (0, 0)),
        out_shape=jax.ShapeDtypeStruct((L, 16), f32),
        scratch_shapes=[pltpu.VMEM((L, 32), f32)],
        compiler_params=pltpu.CompilerParams(
            dimension_semantics=("arbitrary",)),
    )(pair2, node, cax, cay, caz, caxt, cayt, cazt, idxr, idxc,
      W_e1, b_e1.reshape(1, -1),
      W_e2[:32], W_e2[32:68], W_e2[68:69], b_e2.reshape(1, -1),
      W_msg[:32], W_msg[32:64], W_msg[64:96], W_msg[96:99], b_msg.reshape(1, -1),
      W_out[:32], W_out[32:], b_out.reshape(1, -1))

    return out.reshape(1, L, 16)
